# Initial kernel scaffold; baseline (speedup 1.0000x reference)
import jax
import jax.numpy as jnp
from jax.experimental import pallas as pl


def kernel(X, adj, Y, W1, b1, W2, b2, edge_weight):
    raise NotImplementedError("write your pallas kernel here")


# speedup vs baseline: 7.1472x; 7.1472x over previous
import jax
import jax.numpy as jnp
from jax import lax
from jax.experimental import pallas as pl
from jax.experimental.pallas import tpu as pltpu
from jax.experimental.pallas import tpu_sc as plsc

N = 10000
E = 160000
D_IN = 256
D_HID = 256
D_OUT = 64

F1 = 176
F2 = 128
SB = 32
PB1 = 36
PB2 = 54
NSUB = 16
NCORE = 2
EPAD = 165888
NBTOT = EPAD // SB
ROWS_PT = N // NSUB


def _sc_spmm(z0, z1, edata, F, split_edges):
  nvec = F // 16
  if split_edges:
    nb = NBTOT // (2 * NSUB)
    PB = PB2
  else:
    nb = NBTOT // NSUB
    PB = PB1
  nphase = nb // PB

  def body(z0_hbm, z1_hbm, ed_hbm, out0_hbm, out1_hbm,
           acc, rows, ebuf, gs0, gs1, gs2, ss0, ss1, ss2):
    cid = lax.axis_index("c")
    sid = lax.axis_index("s")

    def run(z_hbm, out_hbm):
      if split_edges:
        bbase = (cid * NSUB + sid) * nb
      else:
        bbase = sid * nb
      gsem = (gs0, gs1, gs2)
      ssem = (ss0, ss1, ss2)

      def zrow(j, _):
        for c in range(nvec):
          rows[0, j, pl.ds(c * 16, 16)] = jnp.zeros((16,), jnp.float32)
        return 0
      lax.fori_loop(0, SB, zrow, 0)
      nz = ROWS_PT // SB
      def zcopy(zi, _):
        pltpu.sync_copy(rows.at[0],
                        acc.at[pl.ds(sid * ROWS_PT + zi * SB, SB)])
        return 0
      lax.fori_loop(0, nz, zcopy, 0)
      rem = ROWS_PT - nz * SB
      if rem:
        pltpu.sync_copy(rows.at[0, pl.ds(0, rem)],
                        acc.at[pl.ds(sid * ROWS_PT + nz * SB, rem)])
      plsc.subcore_barrier()

      def g_desc(k, x):
        return pltpu.make_async_copy(z_hbm.at[ebuf.at[k, 0]], rows.at[x],
                                     gsem[x])

      def s_desc(k, x):
        return pltpu.make_async_copy(rows.at[x], acc.at[ebuf.at[k, 1]],
                                     ssem[x])

      def s_start(k, x):
        pltpu.async_copy(rows.at[x], acc.at[ebuf.at[k, 1]], ssem[x],
                         add=True)

      def scale(k, x):
        k16 = jnp.full((16,), k, jnp.int32)
        two16 = jnp.full((16,), 2, jnp.int32)
        def sbody(j, _):
          w16 = plsc.bitcast(
              plsc.load_gather(ebuf, [k16, two16,
                                      jnp.full((16,), j, jnp.int32)]),
              jnp.float32)
          for c in range(nvec):
            sl = pl.ds(c * 16, 16)
            rows[x, j, sl] = rows[x, j, sl] * w16
          return 0
        lax.fori_loop(0, SB, sbody, 0)

      def phase(p, _):
        @pl.when(p > 0)
        def _():
          for x in (0, 1, 2):
            s_desc(0, x).wait()
        pltpu.sync_copy(ed_hbm.at[pl.ds(bbase + p * PB, PB)], ebuf)
        g_desc(0, 0).start()

        def step(t, _):
          for x in (0, 1, 2):
            k = 3 * t + x
            y = (x + 1) % 3
            @pl.when(jnp.logical_and(k >= 2, k <= PB - 2))
            def _():
              s_desc(0, y).wait()
            @pl.when(k <= PB - 2)
            def _():
              g_desc(k + 1, y).start()
            g_desc(k, x).wait()
            scale(k, x)
            s_start(k, x)
          return 0
        lax.fori_loop(0, PB // 3, step, 0)
        return 0
      lax.fori_loop(0, nphase, phase, 0)
      for x in (0, 1, 2):
        s_desc(0, x).wait()
      plsc.subcore_barrier()

      sl = pl.ds(sid * ROWS_PT, ROWS_PT)
      pltpu.sync_copy(acc.at[sl], out_hbm.at[sl])

    @pl.when(cid == 0)
    def _():
      run(z0_hbm, out0_hbm)

    @pl.when(cid == 1)
    def _():
      run(z1_hbm, out1_hbm)

  mesh = plsc.VectorSubcoreMesh(core_axis_name="c", subcore_axis_name="s")
  f = pl.kernel(
      body,
      out_type=[jax.ShapeDtypeStruct((N, F), jnp.float32),
                jax.ShapeDtypeStruct((N, F), jnp.float32)],
      mesh=mesh,
      scratch_types=[
          pltpu.VMEM_SHARED((N, F), jnp.float32),
          pltpu.VMEM((3, SB, F), jnp.float32),
          pltpu.VMEM((PB, 3, SB), jnp.int32),
          pltpu.SemaphoreType.DMA,
          pltpu.SemaphoreType.DMA,
          pltpu.SemaphoreType.DMA,
          pltpu.SemaphoreType.DMA,
          pltpu.SemaphoreType.DMA,
          pltpu.SemaphoreType.DMA,
      ],
      compiler_params=pltpu.CompilerParams(use_tc_tiling_on_sc=False,
                                           needs_layout_passes=False),
  )
  return f(z0, z1, edata)


def _prep_body(x_ref, w1_ref, y_ref, ew_ref, z0_ref, z1_ref, we_ref):
  xw = jnp.dot(x_ref[...], w1_ref[...], preferred_element_type=jnp.float32)
  z0_ref[...] = xw[:, :F1]
  r = xw.shape[0]
  ones = jnp.ones((r, 1), jnp.float32)
  zeros = jnp.zeros((r, F1 - (D_HID - F1) - D_OUT - 1), jnp.float32)
  z1_ref[...] = jnp.concatenate([xw[:, F1:], y_ref[...], ones, zeros], axis=1)
  we_ref[...] = jnp.exp(ew_ref[...])


def _mid_body(p0_ref, p1_ref, w2_ref, b1_ref, z2_ref):
  dn = p1_ref[:, D_HID - F1 + D_OUT:D_HID - F1 + D_OUT + 1] + 1e-16
  pre = jnp.concatenate([p0_ref[...], p1_ref[:, :D_HID - F1]], axis=1)
  h = jnp.maximum(pre / dn + b1_ref[...], 0.0)
  hw2 = jnp.dot(h, w2_ref[...], preferred_element_type=jnp.float32)
  z2_ref[...] = jnp.concatenate(
      [hw2, p1_ref[:, D_HID - F1:D_HID - F1 + D_OUT] / dn], axis=1)


def _log_softmax(o):
  o = o - jnp.max(o, axis=1, keepdims=True)
  return o - jnp.log(jnp.sum(jnp.exp(o), axis=1, keepdims=True))


def _final_body(pa_ref, pb_ref, dn_ref, b2_ref, out_ref, y_ref):
  dn = dn_ref[:, D_HID - F1 + D_OUT:D_HID - F1 + D_OUT + 1] + 1e-16
  p2 = pa_ref[...] + pb_ref[...]
  out_ref[...] = _log_softmax(p2[:, :D_OUT] / dn + b2_ref[...])
  y_ref[...] = _log_softmax(p2[:, D_OUT:] / dn)


def kernel(X, adj, Y, W1, b1, W2, b2, edge_weight):
  src = adj[0]
  dst = adj[1]

  R = 1000
  grid = (N // R,)

  z10, z11, wexp = pl.pallas_call(
      _prep_body,
      grid=grid,
      in_specs=[
          pl.BlockSpec((R, D_IN), lambda i: (i, 0)),
          pl.BlockSpec((D_IN, D_HID), lambda i: (0, 0)),
          pl.BlockSpec((R, D_OUT), lambda i: (i, 0)),
          pl.BlockSpec((8, 2000), lambda i: (i, 0)),
      ],
      out_specs=[
          pl.BlockSpec((R, F1), lambda i: (i, 0)),
          pl.BlockSpec((R, F1), lambda i: (i, 0)),
          pl.BlockSpec((8, 2000), lambda i: (i, 0)),
      ],
      out_shape=[jax.ShapeDtypeStruct((N, F1), jnp.float32),
                 jax.ShapeDtypeStruct((N, F1), jnp.float32),
                 jax.ShapeDtypeStruct((80, 2000), jnp.float32)],
  )(X, W1, Y, edge_weight.reshape(80, 2000))

  pad = EPAD - E
  ar = (jnp.arange(pad, dtype=jnp.int32) * 16) % N
  srcp = jnp.concatenate([src, ar]).reshape(NBTOT, 1, SB)
  dstp = jnp.concatenate([dst, ar]).reshape(NBTOT, 1, SB)
  wbits = lax.bitcast_convert_type(
      jnp.concatenate([wexp.reshape(E), jnp.zeros((pad,), jnp.float32)]),
      jnp.int32).reshape(NBTOT, 1, SB)
  edata = jnp.concatenate([srcp, dstp, wbits], axis=1)

  p10, p11 = _sc_spmm(z10, z11, edata, F1, split_edges=False)

  z2 = pl.pallas_call(
      _mid_body,
      grid=grid,
      in_specs=[
          pl.BlockSpec((R, F1), lambda i: (i, 0)),
          pl.BlockSpec((R, F1), lambda i: (i, 0)),
          pl.BlockSpec((D_HID, D_OUT), lambda i: (0, 0)),
          pl.BlockSpec((1, D_HID), lambda i: (0, 0)),
      ],
      out_specs=pl.BlockSpec((R, F2), lambda i: (i, 0)),
      out_shape=jax.ShapeDtypeStruct((N, F2), jnp.float32),
  )(p10, p11, W2, b1.reshape(1, D_HID))

  p2a, p2b = _sc_spmm(z2, z2, edata, F2, split_edges=True)

  out, y2 = pl.pallas_call(
      _final_body,
      grid=grid,
      in_specs=[
          pl.BlockSpec((R, F2), lambda i: (i, 0)),
          pl.BlockSpec((R, F2), lambda i: (i, 0)),
          pl.BlockSpec((R, F1), lambda i: (i, 0)),
          pl.BlockSpec((1, D_OUT), lambda i: (0, 0)),
      ],
      out_specs=[
          pl.BlockSpec((R, D_OUT), lambda i: (i, 0)),
          pl.BlockSpec((R, D_OUT), lambda i: (i, 0)),
      ],
      out_shape=[jax.ShapeDtypeStruct((N, D_OUT), jnp.float32),
                 jax.ShapeDtypeStruct((N, D_OUT), jnp.float32)],
  )(p2a, p2b, p11, b2.reshape(1, D_OUT))

  return (out, y2)


# speedup vs baseline: 7.6557x; 1.0711x over previous
import jax
import jax.numpy as jnp
from jax import lax
from jax.experimental import pallas as pl
from jax.experimental.pallas import tpu as pltpu
from jax.experimental.pallas import tpu_sc as plsc

N = 10000
E = 160000
D_IN = 256
D_HID = 256
D_OUT = 64

F1 = 176
F2 = 128
SB = 32
PB1 = 36
PB2 = 54
NSUB = 16
NCORE = 2
EPAD = 165888
NBTOT = EPAD // SB
ROWS_PT = N // NSUB


def _sc_spmm(z0, z1, srcb, dstb, wb, F, split_edges):
  nvec = F // 16
  if split_edges:
    nb = NBTOT // (2 * NSUB)
    PB = PB2
  else:
    nb = NBTOT // NSUB
    PB = PB1
  nphase = nb // PB

  def body(z0_hbm, z1_hbm, src_hbm, dst_hbm, w_hbm, out0_hbm, out1_hbm,
           acc, rows, ebuf, gs0, gs1, gs2, ss0, ss1, ss2):
    cid = lax.axis_index("c")
    sid = lax.axis_index("s")

    def run(z_hbm, out_hbm):
      if split_edges:
        bbase = (cid * NSUB + sid) * nb
      else:
        bbase = sid * nb
      gsem = (gs0, gs1, gs2)
      ssem = (ss0, ss1, ss2)

      def zrow(j, _):
        for c in range(nvec):
          rows[0, j, pl.ds(c * 16, 16)] = jnp.zeros((16,), jnp.float32)
        return 0
      lax.fori_loop(0, SB, zrow, 0)
      nz = ROWS_PT // SB
      def zcopy(zi, _):
        pltpu.sync_copy(rows.at[0],
                        acc.at[pl.ds(sid * ROWS_PT + zi * SB, SB)])
        return 0
      lax.fori_loop(0, nz, zcopy, 0)
      rem = ROWS_PT - nz * SB
      if rem:
        pltpu.sync_copy(rows.at[0, pl.ds(0, rem)],
                        acc.at[pl.ds(sid * ROWS_PT + nz * SB, rem)])
      plsc.subcore_barrier()

      def g_desc(k, x):
        return pltpu.make_async_copy(z_hbm.at[ebuf.at[0, k]], rows.at[x],
                                     gsem[x])

      def s_desc(k, x):
        return pltpu.make_async_copy(rows.at[x], acc.at[ebuf.at[1, k]],
                                     ssem[x])

      def s_start(k, x):
        pltpu.async_copy(rows.at[x], acc.at[ebuf.at[1, k]], ssem[x],
                         add=True)

      def scale(k, x):
        def gbody(g, _):
          wvec = plsc.bitcast(ebuf[2, k, pl.ds(g * 16, 16)], jnp.float32)
          dnums = lax.GatherDimensionNumbers(
              offset_dims=(), collapsed_slice_dims=(0,), start_index_map=(0,))
          for j in range(16):
            wj = lax.gather(wvec, jnp.full((16, 1), j, jnp.int32),
                            dimension_numbers=dnums, slice_sizes=(1,),
                            mode=lax.GatherScatterMode.PROMISE_IN_BOUNDS)
            r = g * 16 + j
            for c in range(nvec):
              sl = pl.ds(c * 16, 16)
              rows[x, r, sl] = rows[x, r, sl] * wj
          return 0
        lax.fori_loop(0, SB // 16, gbody, 0)

      def phase(p, _):
        @pl.when(p > 0)
        def _():
          for x in (0, 1, 2):
            s_desc(0, x).wait()
        bsl = pl.ds(bbase + p * PB, PB)
        pltpu.sync_copy(src_hbm.at[bsl], ebuf.at[0])
        pltpu.sync_copy(dst_hbm.at[bsl], ebuf.at[1])
        pltpu.sync_copy(w_hbm.at[bsl], ebuf.at[2])
        g_desc(0, 0).start()

        def step(t, _):
          for x in (0, 1, 2):
            k = 3 * t + x
            y = (x + 1) % 3
            @pl.when(jnp.logical_and(k >= 2, k <= PB - 2))
            def _():
              s_desc(0, y).wait()
            @pl.when(k <= PB - 2)
            def _():
              g_desc(k + 1, y).start()
            g_desc(k, x).wait()
            scale(k, x)
            s_start(k, x)
          return 0
        lax.fori_loop(0, PB // 3, step, 0)
        return 0
      lax.fori_loop(0, nphase, phase, 0)
      for x in (0, 1, 2):
        s_desc(0, x).wait()
      plsc.subcore_barrier()

      sl = pl.ds(sid * ROWS_PT, ROWS_PT)
      pltpu.sync_copy(acc.at[sl], out_hbm.at[sl])

    @pl.when(cid == 0)
    def _():
      run(z0_hbm, out0_hbm)

    @pl.when(cid == 1)
    def _():
      run(z1_hbm, out1_hbm)

  mesh = plsc.VectorSubcoreMesh(core_axis_name="c", subcore_axis_name="s")
  f = pl.kernel(
      body,
      out_type=[jax.ShapeDtypeStruct((N, F), jnp.float32),
                jax.ShapeDtypeStruct((N, F), jnp.float32)],
      mesh=mesh,
      scratch_types=[
          pltpu.VMEM_SHARED((N, F), jnp.float32),
          pltpu.VMEM((3, SB, F), jnp.float32),
          pltpu.VMEM((3, PB, SB), jnp.int32),
          pltpu.SemaphoreType.DMA,
          pltpu.SemaphoreType.DMA,
          pltpu.SemaphoreType.DMA,
          pltpu.SemaphoreType.DMA,
          pltpu.SemaphoreType.DMA,
          pltpu.SemaphoreType.DMA,
      ],
      compiler_params=pltpu.CompilerParams(use_tc_tiling_on_sc=False,
                                           needs_layout_passes=False),
  )
  return f(z0, z1, srcb, dstb, wb)


def _prep_body(x_ref, w1_ref, y_ref, src_ref, dst_ref, ew_ref,
               z0_ref, z1_ref, *sd_ref):
  xw = jnp.dot(x_ref[...], w1_ref[...], preferred_element_type=jnp.float32)
  z0_ref[...] = xw[:, :F1]
  r = xw.shape[0]
  ones = jnp.ones((r, 1), jnp.float32)
  zeros = jnp.zeros((r, F1 - (D_HID - F1) - D_OUT - 1), jnp.float32)
  z1_ref[...] = jnp.concatenate([xw[:, F1:], y_ref[...], ones, zeros], axis=1)

  @pl.when(pl.program_id(0) == 0)
  def _():
    pad = EPAD - E
    ri = lax.broadcasted_iota(jnp.int32, (pad // SB, SB), 0)
    ci = lax.broadcasted_iota(jnp.int32, (pad // SB, SB), 1)
    ar = ((ri * SB + ci) * 16) % N
    sd_ref[0][...] = jnp.concatenate([src_ref[...], ar], axis=0)
    sd_ref[1][...] = jnp.concatenate([dst_ref[...], ar], axis=0)
    wbits = lax.bitcast_convert_type(jnp.exp(ew_ref[...]), jnp.int32)
    zpad = jnp.zeros((pad // SB, SB), jnp.int32)
    sd_ref[2][...] = jnp.concatenate([wbits, zpad], axis=0)


def _mid_body(p0_ref, p1_ref, w2_ref, b1_ref, z2_ref):
  dn = p1_ref[:, D_HID - F1 + D_OUT:D_HID - F1 + D_OUT + 1] + 1e-16
  pre = jnp.concatenate([p0_ref[...], p1_ref[:, :D_HID - F1]], axis=1)
  h = jnp.maximum(pre / dn + b1_ref[...], 0.0)
  hw2 = jnp.dot(h, w2_ref[...], preferred_element_type=jnp.float32)
  z2_ref[...] = jnp.concatenate(
      [hw2, p1_ref[:, D_HID - F1:D_HID - F1 + D_OUT] / dn], axis=1)


def _log_softmax(o):
  o = o - jnp.max(o, axis=1, keepdims=True)
  return o - jnp.log(jnp.sum(jnp.exp(o), axis=1, keepdims=True))


def _final_body(pa_ref, pb_ref, dn_ref, b2_ref, out_ref, y_ref):
  dn = dn_ref[:, D_HID - F1 + D_OUT:D_HID - F1 + D_OUT + 1] + 1e-16
  p2 = pa_ref[...] + pb_ref[...]
  out_ref[...] = _log_softmax(p2[:, :D_OUT] / dn + b2_ref[...])
  y_ref[...] = _log_softmax(p2[:, D_OUT:] / dn)


def kernel(X, adj, Y, W1, b1, W2, b2, edge_weight):
  src = adj[0]
  dst = adj[1]

  R = 1000
  grid = (N // R,)

  z10, z11, srcb, dstb, wb = pl.pallas_call(
      _prep_body,
      grid=grid,
      in_specs=[
          pl.BlockSpec((R, D_IN), lambda i: (i, 0)),
          pl.BlockSpec((D_IN, D_HID), lambda i: (0, 0)),
          pl.BlockSpec((R, D_OUT), lambda i: (i, 0)),
          pl.BlockSpec((E // SB, SB), lambda i: (0, 0)),
          pl.BlockSpec((E // SB, SB), lambda i: (0, 0)),
          pl.BlockSpec((E // SB, SB), lambda i: (0, 0)),
      ],
      out_specs=[
          pl.BlockSpec((R, F1), lambda i: (i, 0)),
          pl.BlockSpec((R, F1), lambda i: (i, 0)),
          pl.BlockSpec((NBTOT, SB), lambda i: (0, 0)),
          pl.BlockSpec((NBTOT, SB), lambda i: (0, 0)),
          pl.BlockSpec((NBTOT, SB), lambda i: (0, 0)),
      ],
      out_shape=[jax.ShapeDtypeStruct((N, F1), jnp.float32),
                 jax.ShapeDtypeStruct((N, F1), jnp.float32),
                 jax.ShapeDtypeStruct((NBTOT, SB), jnp.int32),
                 jax.ShapeDtypeStruct((NBTOT, SB), jnp.int32),
                 jax.ShapeDtypeStruct((NBTOT, SB), jnp.int32)],
  )(X, W1, Y, src.reshape(E // SB, SB), dst.reshape(E // SB, SB),
    edge_weight.reshape(E // SB, SB))

  p10, p11 = _sc_spmm(z10, z11, srcb, dstb, wb, F1, split_edges=False)

  z2 = pl.pallas_call(
      _mid_body,
      grid=grid,
      in_specs=[
          pl.BlockSpec((R, F1), lambda i: (i, 0)),
          pl.BlockSpec((R, F1), lambda i: (i, 0)),
          pl.BlockSpec((D_HID, D_OUT), lambda i: (0, 0)),
          pl.BlockSpec((1, D_HID), lambda i: (0, 0)),
      ],
      out_specs=pl.BlockSpec((R, F2), lambda i: (i, 0)),
      out_shape=jax.ShapeDtypeStruct((N, F2), jnp.float32),
  )(p10, p11, W2, b1.reshape(1, D_HID))

  p2a, p2b = _sc_spmm(z2, z2, srcb, dstb, wb, F2, split_edges=True)

  out, y2 = pl.pallas_call(
      _final_body,
      grid=grid,
      in_specs=[
          pl.BlockSpec((R, F2), lambda i: (i, 0)),
          pl.BlockSpec((R, F2), lambda i: (i, 0)),
          pl.BlockSpec((R, F1), lambda i: (i, 0)),
          pl.BlockSpec((1, D_OUT), lambda i: (0, 0)),
      ],
      out_specs=[
          pl.BlockSpec((R, D_OUT), lambda i: (i, 0)),
          pl.BlockSpec((R, D_OUT), lambda i: (i, 0)),
      ],
      out_shape=[jax.ShapeDtypeStruct((N, D_OUT), jnp.float32),
                 jax.ShapeDtypeStruct((N, D_OUT), jnp.float32)],
  )(p2a, p2b, p11, b2.reshape(1, D_OUT))

  return (out, y2)
